# Initial kernel scaffold; baseline (speedup 1.0000x reference)
import jax
import jax.numpy as jnp
from jax.experimental import pallas as pl


def kernel(product_id, category_id, brand_id, price, is_store_brand, popularity, margin_pct, coupon_clip_rate, coupon_redemption_rate, organic_purchase_ratio, tier_id, elasticity_beta, optimal_discount, discount_offer, product_embed, category_embed, brand_embed, tier_embed, W1, b1, W2, b2):
    raise NotImplementedError("write your pallas kernel here")


# speedup vs baseline: 2.0426x; 2.0426x over previous
import functools

import jax
import jax.numpy as jnp
from jax import lax
from jax.experimental import pallas as pl
from jax.experimental.pallas import tpu as pltpu
from jax.experimental.pallas import tpu_sc as plsc

B = 16384
NC, NS = 2, 16
NW = NC * NS
BPW = B // NW
IDX_CH = 128
NCH = BPW // IDX_CH
HID = 256
OUT = 256
TB = 512


_sc_mesh = plsc.VectorSubcoreMesh(core_axis_name="c", subcore_axis_name="s")


@functools.partial(
    pl.kernel,
    out_type=(
        jax.ShapeDtypeStruct((B, 64), jnp.float32),
        jax.ShapeDtypeStruct((B, 16), jnp.float32),
        jax.ShapeDtypeStruct((B, 16), jnp.float32),
        jax.ShapeDtypeStruct((B, 16), jnp.float32),
    ),
    mesh=_sc_mesh,
    compiler_params=pltpu.CompilerParams(use_tc_tiling_on_sc=False),
    scratch_types=[
        pltpu.VMEM((NCH, IDX_CH), jnp.int32),
        pltpu.VMEM((NCH, IDX_CH), jnp.int32),
        pltpu.VMEM((NCH, IDX_CH), jnp.int32),
        pltpu.VMEM((NCH, IDX_CH), jnp.int32),
        pltpu.VMEM((BPW, 64), jnp.float32),
        pltpu.VMEM((BPW, 16), jnp.float32),
        pltpu.VMEM((BPW, 16), jnp.float32),
        pltpu.VMEM((BPW, 16), jnp.float32),
        pltpu.SemaphoreType.DMA,
    ],
)
def _sc_gather(pid, cid, bid, tid, ptab, ctab, btab, ttab,
               pe_out, ce_out, be_out, te_out,
               pidx, cidx, bidx, tidx, pe_v, ce_v, be_v, te_v, sem):
    wid = lax.axis_index("s") * NC + lax.axis_index("c")
    base = wid * BPW
    pltpu.sync_copy(pid.at[wid], pidx)
    pltpu.sync_copy(cid.at[wid], cidx)
    pltpu.sync_copy(bid.at[wid], bidx)
    pltpu.sync_copy(tid.at[wid], tidx)
    copies = []
    for j in range(NCH):
        sl = pl.ds(j * IDX_CH, IDX_CH)
        copies.append(pltpu.async_copy(ptab.at[pidx.at[j]], pe_v.at[sl], sem))
        copies.append(pltpu.async_copy(ctab.at[cidx.at[j]], ce_v.at[sl], sem))
        copies.append(pltpu.async_copy(btab.at[bidx.at[j]], be_v.at[sl], sem))
        copies.append(pltpu.async_copy(ttab.at[tidx.at[j]], te_v.at[sl], sem))
    for c in copies:
        c.wait()
    rows = pl.ds(base, BPW)
    pltpu.sync_copy(pe_v, pe_out.at[rows])
    pltpu.sync_copy(ce_v, ce_out.at[rows])
    pltpu.sync_copy(be_v, be_out.at[rows])
    pltpu.sync_copy(te_v, te_out.at[rows])


def _tc_mlp(pe_ref, ce_ref, be_ref, te_ref, f_ref,
            w1p_ref, w1c_ref, w1b_ref, w1t_ref, w1f_ref,
            b1_ref, w2_ref, b2_ref, o_ref):
    h = jnp.dot(pe_ref[...], w1p_ref[...], preferred_element_type=jnp.float32)
    h = h + jnp.dot(ce_ref[...], w1c_ref[...],
                    preferred_element_type=jnp.float32)
    h = h + jnp.dot(be_ref[...], w1b_ref[...],
                    preferred_element_type=jnp.float32)
    h = h + jnp.dot(te_ref[...], w1t_ref[...],
                    preferred_element_type=jnp.float32)
    h = h + jnp.dot(f_ref[...], w1f_ref[...],
                    preferred_element_type=jnp.float32)
    h = jnp.maximum(h + b1_ref[...], 0.0)
    y = jnp.dot(h, w2_ref[...], preferred_element_type=jnp.float32) + b2_ref[...]
    n = jnp.sqrt(jnp.sum(y * y, axis=1, keepdims=True))
    o_ref[...] = y / jnp.maximum(n, 1e-12)


_tc_call = pl.pallas_call(
    _tc_mlp,
    grid=(B // TB,),
    in_specs=[
        pl.BlockSpec((TB, 64), lambda i: (i, 0)),
        pl.BlockSpec((TB, 16), lambda i: (i, 0)),
        pl.BlockSpec((TB, 16), lambda i: (i, 0)),
        pl.BlockSpec((TB, 16), lambda i: (i, 0)),
        pl.BlockSpec((TB, 16), lambda i: (i, 0)),
        pl.BlockSpec((64, HID), lambda i: (0, 0)),
        pl.BlockSpec((16, HID), lambda i: (0, 0)),
        pl.BlockSpec((16, HID), lambda i: (0, 0)),
        pl.BlockSpec((16, HID), lambda i: (0, 0)),
        pl.BlockSpec((16, HID), lambda i: (0, 0)),
        pl.BlockSpec((1, HID), lambda i: (0, 0)),
        pl.BlockSpec((HID, OUT), lambda i: (0, 0)),
        pl.BlockSpec((1, OUT), lambda i: (0, 0)),
    ],
    out_specs=pl.BlockSpec((TB, OUT), lambda i: (i, 0)),
    out_shape=jax.ShapeDtypeStruct((B, OUT), jnp.float32),
)


def kernel(product_id, category_id, brand_id, price, is_store_brand,
           popularity, margin_pct, coupon_clip_rate, coupon_redemption_rate,
           organic_purchase_ratio, tier_id, elasticity_beta, optimal_discount,
           discount_offer, product_embed, category_embed, brand_embed,
           tier_embed, W1, b1, W2, b2):
    pid = product_id.astype(jnp.int32).reshape(NW, NCH, IDX_CH)
    cid = category_id.astype(jnp.int32).reshape(NW, NCH, IDX_CH)
    bid = brand_id.astype(jnp.int32).reshape(NW, NCH, IDX_CH)
    tid = tier_id.astype(jnp.int32).reshape(NW, NCH, IDX_CH)
    ttab = jnp.pad(tier_embed, ((0, 0), (0, 8)))

    pe, ce, be, te = _sc_gather(pid, cid, bid, tid, product_embed,
                                category_embed, brand_embed, ttab)

    feats = jnp.stack(
        [price, is_store_brand, popularity, margin_pct, coupon_clip_rate,
         coupon_redemption_rate, organic_purchase_ratio, elasticity_beta,
         optimal_discount, discount_offer], axis=1)
    feats = jnp.pad(feats, ((0, 0), (0, 6)))

    w1p = W1[:, :64].T
    w1c = W1[:, 64:80].T
    w1b = W1[:, 80:96].T
    w1t = jnp.concatenate(
        [W1[:, 103:111], jnp.zeros((HID, 8), jnp.float32)], axis=1).T
    w1f = jnp.concatenate(
        [W1[:, 96:103], W1[:, 111:114], jnp.zeros((HID, 6), jnp.float32)],
        axis=1).T

    return _tc_call(pe, ce, be, te, feats, w1p, w1c, w1b, w1t, w1f,
                    b1.reshape(1, HID), W2.T, b2.reshape(1, OUT))
